# Initial kernel scaffold; baseline (speedup 1.0000x reference)
import jax
import jax.numpy as jnp
from jax.experimental import pallas as pl


def kernel(x, edge_index, revedge_index, edge_attr, batch, W1, W2, W3, b3, fcW, fcb):
    raise NotImplementedError("write your pallas kernel here")


# speedup vs baseline: 1.8301x; 1.8301x over previous
import functools

import jax
import jax.numpy as jnp
from jax import lax
from jax.experimental import pallas as pl
from jax.experimental.pallas import tpu as pltpu
from jax.experimental.pallas import tpu_sc as plsc

_N = 10000
_E = 320000
_NF = 128
_H = 128
_EMB = 64
_B = 64

_NC = 2
_NS = 16
_NW = _NC * _NS
_EPW = _E // _NW
_CH = 80
_NCH = _EPW // _CH
_NP = 10240
_RPS = _NP // _NS


def _sc_mesh():
  return plsc.VectorSubcoreMesh(core_axis_name="c", subcore_axis_name="s")


def _sc_gather(table, idx):
  @functools.partial(
      pl.kernel,
      out_type=jax.ShapeDtypeStruct((_E, _H), jnp.float32),
      mesh=_sc_mesh(),
      scratch_types=[
          pltpu.VMEM((_CH,), jnp.int32),
          pltpu.VMEM((_CH, _H), jnp.float32),
          pltpu.SemaphoreType.DMA,
      ],
  )
  def k(table_hbm, idx_hbm, out_hbm, idx_v, rows_v, sem):
    wid = lax.axis_index("s") * _NC + lax.axis_index("c")

    def body(j, carry):
      off = wid * _EPW + j * _CH
      pltpu.sync_copy(idx_hbm.at[pl.ds(off, _CH)], idx_v)
      pltpu.async_copy(table_hbm.at[idx_v], rows_v, sem).wait()
      pltpu.sync_copy(rows_v, out_hbm.at[pl.ds(off, _CH)])
      return carry

    lax.fori_loop(0, _NCH, body, 0)

  return k(table, idx)


def _sc_scatter_sum(vals, dst, zeros_n):
  @functools.partial(
      pl.kernel,
      out_type=(
          jax.ShapeDtypeStruct((_NP, _H), jnp.float32),
          jax.ShapeDtypeStruct((_NP, _H), jnp.float32),
      ),
      mesh=_sc_mesh(),
      scratch_types=[
          pltpu.VMEM((_CH,), jnp.int32),
          pltpu.VMEM((_CH, _H), jnp.float32),
          pltpu.VMEM_SHARED((_NP, _H), jnp.float32),
          pltpu.SemaphoreType.DMA,
      ],
  )
  def k(vals_hbm, dst_hbm, z_hbm, out0_hbm, out1_hbm, idx_v, rows_v, acc, sem):
    c = lax.axis_index("c")
    s = lax.axis_index("s")
    wid = s * _NC + c
    pltpu.sync_copy(z_hbm.at[pl.ds(s * _RPS, _RPS)],
                    acc.at[pl.ds(s * _RPS, _RPS)])
    plsc.subcore_barrier()

    def body(j, carry):
      off = wid * _EPW + j * _CH
      pltpu.sync_copy(dst_hbm.at[pl.ds(off, _CH)], idx_v)
      pltpu.sync_copy(vals_hbm.at[pl.ds(off, _CH)], rows_v)
      pltpu.sync_copy(rows_v, acc.at[idx_v], add=True)
      return carry

    lax.fori_loop(0, _NCH, body, 0)
    plsc.subcore_barrier()

    @pl.when(c == 0)
    def _():
      pltpu.sync_copy(acc.at[pl.ds(s * _RPS, _RPS)],
                      out0_hbm.at[pl.ds(s * _RPS, _RPS)])

    @pl.when(c == 1)
    def _():
      pltpu.sync_copy(acc.at[pl.ds(s * _RPS, _RPS)],
                      out1_hbm.at[pl.ds(s * _RPS, _RPS)])

  return k(vals, dst, zeros_n)


_BE = 2560
_GE = _E // _BE


def _tc_h0_g0(xs, ea, w1xt, w1e, w2t):
  def body(xs_ref, ea_ref, w1xt_ref, w1e_ref, w2t_ref, h0_ref, g0_ref):
    h0 = jnp.maximum(
        jnp.dot(xs_ref[...], w1xt_ref[...], preferred_element_type=jnp.float32)
        + ea_ref[...] * w1e_ref[...], 0.0)
    h0_ref[...] = h0
    g0_ref[...] = jnp.dot(h0, w2t_ref[...], preferred_element_type=jnp.float32)

  return pl.pallas_call(
      body,
      grid=(_GE,),
      in_specs=[
          pl.BlockSpec((_BE, _NF), lambda i: (i, 0)),
          pl.BlockSpec((_BE, 1), lambda i: (i, 0)),
          pl.BlockSpec((_NF, _H), lambda i: (0, 0)),
          pl.BlockSpec((1, _H), lambda i: (0, 0)),
          pl.BlockSpec((_H, _H), lambda i: (0, 0)),
      ],
      out_specs=[
          pl.BlockSpec((_BE, _H), lambda i: (i, 0)),
          pl.BlockSpec((_BE, _H), lambda i: (i, 0)),
      ],
      out_shape=[jax.ShapeDtypeStruct((_E, _H), jnp.float32)] * 2,
  )(xs, ea, w1xt, w1e, w2t)


def _tc_layer(h0, ns, gr, w2t):
  def body(h0_ref, ns_ref, gr_ref, w2t_ref, g_ref):
    h = jnp.maximum(h0_ref[...] + ns_ref[...] - gr_ref[...], 0.0)
    g_ref[...] = jnp.dot(h, w2t_ref[...], preferred_element_type=jnp.float32)

  return pl.pallas_call(
      body,
      grid=(_GE,),
      in_specs=[
          pl.BlockSpec((_BE, _H), lambda i: (i, 0)),
          pl.BlockSpec((_BE, _H), lambda i: (i, 0)),
          pl.BlockSpec((_BE, _H), lambda i: (i, 0)),
          pl.BlockSpec((_H, _H), lambda i: (0, 0)),
      ],
      out_specs=pl.BlockSpec((_BE, _H), lambda i: (i, 0)),
      out_shape=jax.ShapeDtypeStruct((_E, _H), jnp.float32),
  )(h0, ns, gr, w2t)


def _tc_relu3(h0, ns, gr):
  def body(h0_ref, ns_ref, gr_ref, h_ref):
    h_ref[...] = jnp.maximum(h0_ref[...] + ns_ref[...] - gr_ref[...], 0.0)

  return pl.pallas_call(
      body,
      grid=(_GE,),
      in_specs=[
          pl.BlockSpec((_BE, _H), lambda i: (i, 0)),
          pl.BlockSpec((_BE, _H), lambda i: (i, 0)),
          pl.BlockSpec((_BE, _H), lambda i: (i, 0)),
      ],
      out_specs=pl.BlockSpec((_BE, _H), lambda i: (i, 0)),
      out_shape=jax.ShapeDtypeStruct((_E, _H), jnp.float32),
  )(h0, ns, gr)


_BN = 1280
_GN = _NP // _BN


def _tc_add(a, b):
  def body(a_ref, b_ref, o_ref):
    o_ref[...] = a_ref[...] + b_ref[...]

  return pl.pallas_call(
      body,
      grid=(_GN,),
      in_specs=[
          pl.BlockSpec((_BN, _H), lambda i: (i, 0)),
          pl.BlockSpec((_BN, _H), lambda i: (i, 0)),
      ],
      out_specs=pl.BlockSpec((_BN, _H), lambda i: (i, 0)),
      out_shape=jax.ShapeDtypeStruct((_NP, _H), jnp.float32),
  )(a, b)


_BF = 1000
_GF = _N // _BF


def _tc_final(x, vp0, vp1, batch2d, w3xt, w3mt, b3row, fcwt, fcbrow):
  def body(x_ref, vp0_ref, vp1_ref, b_ref, w3xt_ref, w3mt_ref, b3_ref,
           fcwt_ref, fcb_ref, out_ref, acc_s, acc_c):
    i = pl.program_id(0)
    vmsg = vp0_ref[...] + vp1_ref[...]
    na = jnp.maximum(
        jnp.dot(x_ref[...], w3xt_ref[...], preferred_element_type=jnp.float32)
        + jnp.dot(vmsg, w3mt_ref[...], preferred_element_type=jnp.float32)
        + b3_ref[...], 0.0)
    cols = lax.broadcasted_iota(jnp.int32, (_BF, _B), 1)
    oh = (b_ref[...] == cols).astype(jnp.float32)
    ps = lax.dot_general(oh, na, (((0,), (0,)), ((), ())),
                         preferred_element_type=jnp.float32)
    pc = lax.dot_general(oh, jnp.ones((_BF, _H), jnp.float32),
                         (((0,), (0,)), ((), ())),
                         preferred_element_type=jnp.float32)

    @pl.when(i == 0)
    def _():
      acc_s[...] = jnp.zeros((_B, _H), jnp.float32)
      acc_c[...] = jnp.zeros((_B, _H), jnp.float32)

    acc_s[...] += ps
    acc_c[...] += pc

    @pl.when(i == _GF - 1)
    def _():
      pooled = acc_s[...] / jnp.maximum(acc_c[...], 1.0)
      out_ref[...] = jnp.tanh(
          jnp.dot(pooled, fcwt_ref[...], preferred_element_type=jnp.float32)
          + fcb_ref[...])

  return pl.pallas_call(
      body,
      grid=(_GF,),
      in_specs=[
          pl.BlockSpec((_BF, _NF), lambda i: (i, 0)),
          pl.BlockSpec((_BF, _H), lambda i: (i, 0)),
          pl.BlockSpec((_BF, _H), lambda i: (i, 0)),
          pl.BlockSpec((_BF, 1), lambda i: (i, 0)),
          pl.BlockSpec((_NF, _H), lambda i: (0, 0)),
          pl.BlockSpec((_H, _H), lambda i: (0, 0)),
          pl.BlockSpec((1, _H), lambda i: (0, 0)),
          pl.BlockSpec((_H, _EMB), lambda i: (0, 0)),
          pl.BlockSpec((1, _EMB), lambda i: (0, 0)),
      ],
      out_specs=pl.BlockSpec((_B, _EMB), lambda i: (0, 0)),
      out_shape=jax.ShapeDtypeStruct((_B, _EMB), jnp.float32),
      scratch_shapes=[
          pltpu.VMEM((_B, _H), jnp.float32),
          pltpu.VMEM((_B, _H), jnp.float32),
      ],
  )(x, vp0, vp1, batch2d, w3xt, w3mt, b3row, fcwt, fcbrow)


def kernel(x, edge_index, revedge_index, edge_attr, batch, W1, W2, W3, b3,
           fcW, fcb):
  src = edge_index[0]
  dst = edge_index[1]
  ea = edge_attr[:, None]
  w1xt = W1[:, :_NF].T
  w1e = W1[:, _NF][None, :]
  w2t = W2.T
  w3xt = W3[:, :_NF].T
  w3mt = W3[:, _NF:].T
  b3row = b3[None, :]
  fcwt = fcW.T
  fcbrow = fcb[None, :]
  zeros_n = jnp.zeros((_NP, _H), jnp.float32)
  batch2d = batch[:, None]

  xs = _sc_gather(x, src)
  h0, g0 = _tc_h0_g0(xs, ea, w1xt, w1e, w2t)

  p0a, p0b = _sc_scatter_sum(g0, dst, zeros_n)
  n0 = _tc_add(p0a, p0b)
  gr0 = _sc_gather(g0, revedge_index)
  ns0 = _sc_gather(n0, src)
  g1 = _tc_layer(h0, ns0, gr0, w2t)

  p1a, p1b = _sc_scatter_sum(g1, dst, zeros_n)
  n1 = _tc_add(p1a, p1b)
  gr1 = _sc_gather(g1, revedge_index)
  ns1 = _sc_gather(n1, src)
  h2 = _tc_relu3(h0, ns1, gr1)

  vpa, vpb = _sc_scatter_sum(h2, dst, zeros_n)
  return _tc_final(x, vpa, vpb, batch2d, w3xt, w3mt, b3row, fcwt, fcbrow)


# speedup vs baseline: 3.1430x; 1.7174x over previous
import functools

import jax
import jax.numpy as jnp
from jax import lax
from jax.experimental import pallas as pl
from jax.experimental.pallas import tpu as pltpu
from jax.experimental.pallas import tpu_sc as plsc

_N = 10000
_E = 320000
_NF = 128
_H = 128
_EMB = 64
_B = 64

_NC = 2
_NS = 16
_NW = _NC * _NS
_EPW = _E // _NW
_CH = 80
_NCH = _EPW // _CH
_NPAIR = (_NCH - 1) // 2
_NP = 10240
_RPS = _NP // _NS
_NCOL = _H // 16


def _sc_mesh():
  return plsc.VectorSubcoreMesh(core_axis_name="c", subcore_axis_name="s")


def _worker_id():
  return lax.axis_index("s") * _NC + lax.axis_index("c")


def _sc_gather(table, idx):
  @functools.partial(
      pl.kernel,
      out_type=jax.ShapeDtypeStruct((_E, _H), jnp.float32),
      mesh=_sc_mesh(),
      scratch_types=[
          pltpu.VMEM((_EPW,), jnp.int32),
          pltpu.VMEM((_CH, _H), jnp.float32),
          pltpu.VMEM((_CH, _H), jnp.float32),
          pltpu.SemaphoreType.DMA,
          pltpu.SemaphoreType.DMA,
          pltpu.SemaphoreType.DMA,
          pltpu.SemaphoreType.DMA,
      ],
  )
  def k(table_hbm, idx_hbm, out_hbm, idx_all, r0, r1, g0, g1, s0, s1):
    base = _worker_id() * _EPW
    pltpu.sync_copy(idx_hbm.at[pl.ds(base, _EPW)], idx_all)

    def issue(j, rows, gsem):
      pltpu.async_copy(table_hbm.at[idx_all.at[pl.ds(j * _CH, _CH)]],
                       rows, gsem)

    def wait_g(rows, gsem):
      pltpu.make_async_copy(table_hbm.at[idx_all.at[pl.ds(0, _CH)]],
                            rows, gsem).wait()

    def store(j, rows, ssem):
      pltpu.async_copy(rows, out_hbm.at[pl.ds(base + j * _CH, _CH)], ssem)

    def wait_s(rows, ssem):
      pltpu.make_async_copy(rows, out_hbm.at[pl.ds(base, _CH)], ssem).wait()

    issue(0, r0, g0)

    def body(t, carry):
      j0 = 2 * t
      j1 = j0 + 1
      j2 = j0 + 2

      @pl.when(t > 0)
      def _():
        wait_s(r1, s1)

      issue(j1, r1, g1)
      wait_g(r0, g0)
      store(j0, r0, s0)
      wait_s(r0, s0)
      issue(j2, r0, g0)
      wait_g(r1, g1)
      store(j1, r1, s1)
      return carry

    lax.fori_loop(0, _NPAIR, body, 0)
    wait_g(r0, g0)
    pltpu.sync_copy(r0, out_hbm.at[pl.ds(base + (_NCH - 1) * _CH, _CH)])
    wait_s(r1, s1)

  return k(table, idx)


def _sc_scatter_sum(vals, dst, zeros_n):
  @functools.partial(
      pl.kernel,
      out_type=(
          jax.ShapeDtypeStruct((_NP, _H), jnp.float32),
          jax.ShapeDtypeStruct((_NP, _H), jnp.float32),
      ),
      mesh=_sc_mesh(),
      scratch_types=[
          pltpu.VMEM((_CH,), jnp.int32),
          pltpu.VMEM((_CH,), jnp.int32),
          pltpu.VMEM((_CH, _H), jnp.float32),
          pltpu.VMEM((_CH, _H), jnp.float32),
          pltpu.VMEM_SHARED((_NP, _H), jnp.float32),
          pltpu.SemaphoreType.DMA,
          pltpu.SemaphoreType.DMA,
      ],
  )
  def k(vals_hbm, dst_hbm, z_hbm, out0_hbm, out1_hbm,
        i0, i1, r0, r1, acc, v0, v1):
    c = lax.axis_index("c")
    s = lax.axis_index("s")
    base = (s * _NC + c) * _EPW
    pltpu.sync_copy(z_hbm.at[pl.ds(s * _RPS, _RPS)],
                    acc.at[pl.ds(s * _RPS, _RPS)])
    plsc.subcore_barrier()

    def issue(j, iv, rows, vsem):
      pltpu.sync_copy(dst_hbm.at[pl.ds(base + j * _CH, _CH)], iv)
      pltpu.async_copy(vals_hbm.at[pl.ds(base + j * _CH, _CH)], rows, vsem)

    def drain(iv, rows, vsem):
      pltpu.make_async_copy(vals_hbm.at[pl.ds(base, _CH)], rows, vsem).wait()
      pltpu.sync_copy(rows, acc.at[iv], add=True)

    issue(0, i0, r0, v0)

    def body(t, carry):
      j1 = 2 * t + 1
      j2 = j1 + 1
      issue(j1, i1, r1, v1)
      drain(i0, r0, v0)
      issue(j2, i0, r0, v0)
      drain(i1, r1, v1)
      return carry

    lax.fori_loop(0, _NPAIR, body, 0)
    drain(i0, r0, v0)
    plsc.subcore_barrier()

    @pl.when(c == 0)
    def _():
      pltpu.sync_copy(acc.at[pl.ds(s * _RPS, _RPS)],
                      out0_hbm.at[pl.ds(s * _RPS, _RPS)])

    @pl.when(c == 1)
    def _():
      pltpu.sync_copy(acc.at[pl.ds(s * _RPS, _RPS)],
                      out1_hbm.at[pl.ds(s * _RPS, _RPS)])

  return k(vals, dst, zeros_n)


def _sc_gather_sub(ntab, g, src, rev):
  @functools.partial(
      pl.kernel,
      out_type=jax.ShapeDtypeStruct((_E, _H), jnp.float32),
      mesh=_sc_mesh(),
      scratch_types=[
          pltpu.VMEM((_EPW,), jnp.int32),
          pltpu.VMEM((_EPW,), jnp.int32),
          pltpu.VMEM((_CH, _H), jnp.float32),
          pltpu.VMEM((_CH, _H), jnp.float32),
          pltpu.VMEM((_CH, _H), jnp.float32),
          pltpu.VMEM((_CH, _H), jnp.float32),
          pltpu.VMEM((_CH, _H), jnp.float32),
          pltpu.VMEM((_CH, _H), jnp.float32),
          pltpu.SemaphoreType.DMA,
          pltpu.SemaphoreType.DMA,
          pltpu.SemaphoreType.DMA,
          pltpu.SemaphoreType.DMA,
      ],
  )
  def k(ntab_hbm, g_hbm, src_hbm, rev_hbm, out_hbm,
        src_all, rev_all, ns0, gr0, ns1, gr1, d0, d1, g0, g1, s0, s1):
    base = _worker_id() * _EPW
    pltpu.sync_copy(src_hbm.at[pl.ds(base, _EPW)], src_all)
    pltpu.sync_copy(rev_hbm.at[pl.ds(base, _EPW)], rev_all)

    def issue(j, nsb, grb, gsem):
      pltpu.async_copy(ntab_hbm.at[src_all.at[pl.ds(j * _CH, _CH)]],
                       nsb, gsem)
      pltpu.async_copy(g_hbm.at[rev_all.at[pl.ds(j * _CH, _CH)]], grb, gsem)

    def wait_g(nsb, grb, gsem):
      pltpu.make_async_copy(ntab_hbm.at[src_all.at[pl.ds(0, _CH)]],
                            nsb, gsem).wait()
      pltpu.make_async_copy(g_hbm.at[rev_all.at[pl.ds(0, _CH)]],
                            grb, gsem).wait()

    def alu(nsb, grb, db):
      def row(r, carry):
        for cc in range(_NCOL):
          sl = pl.ds(cc * 16, 16)
          db[r, sl] = nsb[r, sl] - grb[r, sl]
        return carry
      lax.fori_loop(0, _CH, row, 0)

    def wait_s(db, ssem):
      pltpu.make_async_copy(db, out_hbm.at[pl.ds(base, _CH)], ssem).wait()

    issue(0, ns0, gr0, g0)

    def body(t, carry):
      j0 = 2 * t
      j1 = j0 + 1
      j2 = j0 + 2
      issue(j1, ns1, gr1, g1)
      wait_g(ns0, gr0, g0)

      @pl.when(t > 0)
      def _():
        wait_s(d0, s0)

      alu(ns0, gr0, d0)
      pltpu.async_copy(d0, out_hbm.at[pl.ds(base + j0 * _CH, _CH)], s0)
      issue(j2, ns0, gr0, g0)
      wait_g(ns1, gr1, g1)

      @pl.when(t > 0)
      def _():
        wait_s(d1, s1)

      alu(ns1, gr1, d1)
      pltpu.async_copy(d1, out_hbm.at[pl.ds(base + j1 * _CH, _CH)], s1)
      return carry

    lax.fori_loop(0, _NPAIR, body, 0)
    wait_g(ns0, gr0, g0)
    wait_s(d0, s0)
    alu(ns0, gr0, d0)
    pltpu.sync_copy(d0, out_hbm.at[pl.ds(base + (_NCH - 1) * _CH, _CH)])
    wait_s(d1, s1)

  return k(ntab, g, src, rev)


_CHF = 40
_NCHF = _EPW // _CHF
_NPF = _NCHF // 2


def _sc_final_msg(ntab, g, h0, src, rev, dst, zeros_n):
  @functools.partial(
      pl.kernel,
      out_type=(
          jax.ShapeDtypeStruct((_NP, _H), jnp.float32),
          jax.ShapeDtypeStruct((_NP, _H), jnp.float32),
      ),
      mesh=_sc_mesh(),
      scratch_types=[
          pltpu.VMEM((_CHF,), jnp.int32),
          pltpu.VMEM((_CHF,), jnp.int32),
          pltpu.VMEM((_CHF,), jnp.int32),
          pltpu.VMEM((_CHF,), jnp.int32),
          pltpu.VMEM((_CHF,), jnp.int32),
          pltpu.VMEM((_CHF,), jnp.int32),
          pltpu.VMEM((_CHF, _H), jnp.float32),
          pltpu.VMEM((_CHF, _H), jnp.float32),
          pltpu.VMEM((_CHF, _H), jnp.float32),
          pltpu.VMEM((_CHF, _H), jnp.float32),
          pltpu.VMEM((_CHF, _H), jnp.float32),
          pltpu.VMEM((_CHF, _H), jnp.float32),
          pltpu.VMEM((_CHF, _H), jnp.float32),
          pltpu.VMEM_SHARED((_NP, _H), jnp.float32),
          pltpu.SemaphoreType.DMA,
          pltpu.SemaphoreType.DMA,
      ],
  )
  def k(ntab_hbm, g_hbm, h0_hbm, src_hbm, rev_hbm, dst_hbm, z_hbm,
        out0_hbm, out1_hbm,
        si0, si1, ri0, ri1, di0, di1, ns0, gr0, hb0, ns1, gr1, hb1, mb,
        acc, g0, g1):
    c = lax.axis_index("c")
    s = lax.axis_index("s")
    base = (s * _NC + c) * _EPW
    pltpu.sync_copy(z_hbm.at[pl.ds(s * _RPS, _RPS)],
                    acc.at[pl.ds(s * _RPS, _RPS)])
    plsc.subcore_barrier()

    def issue(j, si, ri, di, nsb, grb, hb, gsem):
      off = base + j * _CHF
      pltpu.sync_copy(src_hbm.at[pl.ds(off, _CHF)], si)
      pltpu.sync_copy(rev_hbm.at[pl.ds(off, _CHF)], ri)
      pltpu.sync_copy(dst_hbm.at[pl.ds(off, _CHF)], di)
      pltpu.async_copy(ntab_hbm.at[si], nsb, gsem)
      pltpu.async_copy(g_hbm.at[ri], grb, gsem)
      pltpu.async_copy(h0_hbm.at[pl.ds(off, _CHF)], hb, gsem)

    def drain(si, ri, di, nsb, grb, hb, gsem):
      pltpu.make_async_copy(ntab_hbm.at[si], nsb, gsem).wait()
      pltpu.make_async_copy(g_hbm.at[ri], grb, gsem).wait()
      pltpu.make_async_copy(h0_hbm.at[pl.ds(base, _CHF)], hb, gsem).wait()

      def row(r, carry):
        for cc in range(_NCOL):
          sl = pl.ds(cc * 16, 16)
          mb[r, sl] = jnp.maximum(hb[r, sl] + nsb[r, sl] - grb[r, sl], 0.0)
        return carry
      lax.fori_loop(0, _CHF, row, 0)
      pltpu.sync_copy(mb, acc.at[di], add=True)

    issue(0, si0, ri0, di0, ns0, gr0, hb0, g0)

    def body(t, carry):
      j1 = 2 * t + 1
      j2 = j1 + 1
      issue(j1, si1, ri1, di1, ns1, gr1, hb1, g1)
      drain(si0, ri0, di0, ns0, gr0, hb0, g0)

      @pl.when(j2 < _NCHF)
      def _():
        issue(j2, si0, ri0, di0, ns0, gr0, hb0, g0)

      drain(si1, ri1, di1, ns1, gr1, hb1, g1)
      return carry

    lax.fori_loop(0, _NPF, body, 0)
    plsc.subcore_barrier()

    @pl.when(c == 0)
    def _():
      pltpu.sync_copy(acc.at[pl.ds(s * _RPS, _RPS)],
                      out0_hbm.at[pl.ds(s * _RPS, _RPS)])

    @pl.when(c == 1)
    def _():
      pltpu.sync_copy(acc.at[pl.ds(s * _RPS, _RPS)],
                      out1_hbm.at[pl.ds(s * _RPS, _RPS)])

  return k(ntab, g, h0, src, rev, dst, zeros_n)


_BE = 2560
_GE = _E // _BE


def _tc_h0_g0(xs, ea, w1xt, w1e, w2t):
  def body(xs_ref, ea_ref, w1xt_ref, w1e_ref, w2t_ref, h0_ref, g0_ref):
    h0 = jnp.maximum(
        jnp.dot(xs_ref[...], w1xt_ref[...], preferred_element_type=jnp.float32)
        + ea_ref[...] * w1e_ref[...], 0.0)
    h0_ref[...] = h0
    g0_ref[...] = jnp.dot(h0, w2t_ref[...], preferred_element_type=jnp.float32)

  return pl.pallas_call(
      body,
      grid=(_GE,),
      in_specs=[
          pl.BlockSpec((_BE, _NF), lambda i: (i, 0)),
          pl.BlockSpec((_BE, 1), lambda i: (i, 0)),
          pl.BlockSpec((_NF, _H), lambda i: (0, 0)),
          pl.BlockSpec((1, _H), lambda i: (0, 0)),
          pl.BlockSpec((_H, _H), lambda i: (0, 0)),
      ],
      out_specs=[
          pl.BlockSpec((_BE, _H), lambda i: (i, 0)),
          pl.BlockSpec((_BE, _H), lambda i: (i, 0)),
      ],
      out_shape=[jax.ShapeDtypeStruct((_E, _H), jnp.float32)] * 2,
  )(xs, ea, w1xt, w1e, w2t)


def _tc_layer(h0, d, w2t):
  def body(h0_ref, d_ref, w2t_ref, g_ref):
    h = jnp.maximum(h0_ref[...] + d_ref[...], 0.0)
    g_ref[...] = jnp.dot(h, w2t_ref[...], preferred_element_type=jnp.float32)

  return pl.pallas_call(
      body,
      grid=(_GE,),
      in_specs=[
          pl.BlockSpec((_BE, _H), lambda i: (i, 0)),
          pl.BlockSpec((_BE, _H), lambda i: (i, 0)),
          pl.BlockSpec((_H, _H), lambda i: (0, 0)),
      ],
      out_specs=pl.BlockSpec((_BE, _H), lambda i: (i, 0)),
      out_shape=jax.ShapeDtypeStruct((_E, _H), jnp.float32),
  )(h0, d, w2t)


_BN = 1280
_GN = _NP // _BN


def _tc_add(a, b):
  def body(a_ref, b_ref, o_ref):
    o_ref[...] = a_ref[...] + b_ref[...]

  return pl.pallas_call(
      body,
      grid=(_GN,),
      in_specs=[
          pl.BlockSpec((_BN, _H), lambda i: (i, 0)),
          pl.BlockSpec((_BN, _H), lambda i: (i, 0)),
      ],
      out_specs=pl.BlockSpec((_BN, _H), lambda i: (i, 0)),
      out_shape=jax.ShapeDtypeStruct((_NP, _H), jnp.float32),
  )(a, b)


_BF = 1000
_GF = _N // _BF


def _tc_final(x, vp0, vp1, batch2d, w3xt, w3mt, b3row, fcwt, fcbrow):
  def body(x_ref, vp0_ref, vp1_ref, b_ref, w3xt_ref, w3mt_ref, b3_ref,
           fcwt_ref, fcb_ref, out_ref, acc_s, acc_c):
    i = pl.program_id(0)
    vmsg = vp0_ref[...] + vp1_ref[...]
    na = jnp.maximum(
        jnp.dot(x_ref[...], w3xt_ref[...], preferred_element_type=jnp.float32)
        + jnp.dot(vmsg, w3mt_ref[...], preferred_element_type=jnp.float32)
        + b3_ref[...], 0.0)
    cols = lax.broadcasted_iota(jnp.int32, (_BF, _B), 1)
    oh = (b_ref[...] == cols).astype(jnp.float32)
    ps = lax.dot_general(oh, na, (((0,), (0,)), ((), ())),
                         preferred_element_type=jnp.float32)
    pc = lax.dot_general(oh, jnp.ones((_BF, _H), jnp.float32),
                         (((0,), (0,)), ((), ())),
                         preferred_element_type=jnp.float32)

    @pl.when(i == 0)
    def _():
      acc_s[...] = jnp.zeros((_B, _H), jnp.float32)
      acc_c[...] = jnp.zeros((_B, _H), jnp.float32)

    acc_s[...] += ps
    acc_c[...] += pc

    @pl.when(i == _GF - 1)
    def _():
      pooled = acc_s[...] / jnp.maximum(acc_c[...], 1.0)
      out_ref[...] = jnp.tanh(
          jnp.dot(pooled, fcwt_ref[...], preferred_element_type=jnp.float32)
          + fcb_ref[...])

  return pl.pallas_call(
      body,
      grid=(_GF,),
      in_specs=[
          pl.BlockSpec((_BF, _NF), lambda i: (i, 0)),
          pl.BlockSpec((_BF, _H), lambda i: (i, 0)),
          pl.BlockSpec((_BF, _H), lambda i: (i, 0)),
          pl.BlockSpec((_BF, 1), lambda i: (i, 0)),
          pl.BlockSpec((_NF, _H), lambda i: (0, 0)),
          pl.BlockSpec((_H, _H), lambda i: (0, 0)),
          pl.BlockSpec((1, _H), lambda i: (0, 0)),
          pl.BlockSpec((_H, _EMB), lambda i: (0, 0)),
          pl.BlockSpec((1, _EMB), lambda i: (0, 0)),
      ],
      out_specs=pl.BlockSpec((_B, _EMB), lambda i: (0, 0)),
      out_shape=jax.ShapeDtypeStruct((_B, _EMB), jnp.float32),
      scratch_shapes=[
          pltpu.VMEM((_B, _H), jnp.float32),
          pltpu.VMEM((_B, _H), jnp.float32),
      ],
  )(x, vp0, vp1, batch2d, w3xt, w3mt, b3row, fcwt, fcbrow)


def kernel(x, edge_index, revedge_index, edge_attr, batch, W1, W2, W3, b3,
           fcW, fcb):
  src = edge_index[0]
  dst = edge_index[1]
  ea = edge_attr[:, None]
  w1xt = W1[:, :_NF].T
  w1e = W1[:, _NF][None, :]
  w2t = W2.T
  w3xt = W3[:, :_NF].T
  w3mt = W3[:, _NF:].T
  b3row = b3[None, :]
  fcwt = fcW.T
  fcbrow = fcb[None, :]
  zeros_n = jnp.zeros((_NP, _H), jnp.float32)
  batch2d = batch[:, None]

  xs = _sc_gather(x, src)
  h0, g0 = _tc_h0_g0(xs, ea, w1xt, w1e, w2t)

  p0a, p0b = _sc_scatter_sum(g0, dst, zeros_n)
  n0 = _tc_add(p0a, p0b)
  d0 = _sc_gather_sub(n0, g0, src, revedge_index)
  g1 = _tc_layer(h0, d0, w2t)

  p1a, p1b = _sc_scatter_sum(g1, dst, zeros_n)
  n1 = _tc_add(p1a, p1b)
  vpa, vpb = _sc_final_msg(n1, g1, h0, src, revedge_index, dst, zeros_n)
  return _tc_final(x, vpa, vpb, batch2d, w3xt, w3mt, b3row, fcwt, fcbrow)


# speedup vs baseline: 3.2975x; 1.0492x over previous
import functools

import jax
import jax.numpy as jnp
from jax import lax
from jax.experimental import pallas as pl
from jax.experimental.pallas import tpu as pltpu
from jax.experimental.pallas import tpu_sc as plsc

_N = 10000
_E = 320000
_NF = 128
_H = 128
_EMB = 64
_B = 64

_NC = 2
_NS = 16
_NW = _NC * _NS
_EPW = _E // _NW
_CH = 80
_NCH = _EPW // _CH
_NPAIR = (_NCH - 1) // 2
_NP = 10240
_RPS = _NP // _NS
_NCOL = _H // 16


def _sc_mesh():
  return plsc.VectorSubcoreMesh(core_axis_name="c", subcore_axis_name="s")


def _worker_id():
  return lax.axis_index("s") * _NC + lax.axis_index("c")


def _sc_gather(table, idx):
  @functools.partial(
      pl.kernel,
      out_type=jax.ShapeDtypeStruct((_E, _H), jnp.float32),
      mesh=_sc_mesh(),
      scratch_types=[
          pltpu.VMEM((_EPW,), jnp.int32),
          pltpu.VMEM((_CH, _H), jnp.float32),
          pltpu.VMEM((_CH, _H), jnp.float32),
          pltpu.SemaphoreType.DMA,
          pltpu.SemaphoreType.DMA,
          pltpu.SemaphoreType.DMA,
          pltpu.SemaphoreType.DMA,
      ],
  )
  def k(table_hbm, idx_hbm, out_hbm, idx_all, r0, r1, g0, g1, s0, s1):
    base = _worker_id() * _EPW
    pltpu.sync_copy(idx_hbm.at[pl.ds(base, _EPW)], idx_all)

    def issue(j, rows, gsem):
      pltpu.async_copy(table_hbm.at[idx_all.at[pl.ds(j * _CH, _CH)]],
                       rows, gsem)

    def wait_g(rows, gsem):
      pltpu.make_async_copy(table_hbm.at[idx_all.at[pl.ds(0, _CH)]],
                            rows, gsem).wait()

    def store(j, rows, ssem):
      pltpu.async_copy(rows, out_hbm.at[pl.ds(base + j * _CH, _CH)], ssem)

    def wait_s(rows, ssem):
      pltpu.make_async_copy(rows, out_hbm.at[pl.ds(base, _CH)], ssem).wait()

    issue(0, r0, g0)

    def body(t, carry):
      j0 = 2 * t
      j1 = j0 + 1
      j2 = j0 + 2

      @pl.when(t > 0)
      def _():
        wait_s(r1, s1)

      issue(j1, r1, g1)
      wait_g(r0, g0)
      store(j0, r0, s0)
      wait_s(r0, s0)
      issue(j2, r0, g0)
      wait_g(r1, g1)
      store(j1, r1, s1)
      return carry

    lax.fori_loop(0, _NPAIR, body, 0)
    wait_g(r0, g0)
    pltpu.sync_copy(r0, out_hbm.at[pl.ds(base + (_NCH - 1) * _CH, _CH)])
    wait_s(r1, s1)

  return k(table, idx)


def _sc_scatter_sum(vals, dst, zeros_n):
  @functools.partial(
      pl.kernel,
      out_type=(
          jax.ShapeDtypeStruct((_NP, _H), jnp.float32),
          jax.ShapeDtypeStruct((_NP, _H), jnp.float32),
      ),
      mesh=_sc_mesh(),
      scratch_types=[
          pltpu.VMEM((_CH,), jnp.int32),
          pltpu.VMEM((_CH,), jnp.int32),
          pltpu.VMEM((_CH, _H), jnp.float32),
          pltpu.VMEM((_CH, _H), jnp.float32),
          pltpu.VMEM_SHARED((_NP, _H), jnp.float32),
          pltpu.SemaphoreType.DMA,
          pltpu.SemaphoreType.DMA,
      ],
  )
  def k(vals_hbm, dst_hbm, z_hbm, out0_hbm, out1_hbm,
        i0, i1, r0, r1, acc, v0, v1):
    c = lax.axis_index("c")
    s = lax.axis_index("s")
    base = (s * _NC + c) * _EPW
    pltpu.sync_copy(z_hbm.at[pl.ds(s * _RPS, _RPS)],
                    acc.at[pl.ds(s * _RPS, _RPS)])
    plsc.subcore_barrier()

    def issue(j, iv, rows, vsem):
      pltpu.sync_copy(dst_hbm.at[pl.ds(base + j * _CH, _CH)], iv)
      pltpu.async_copy(vals_hbm.at[pl.ds(base + j * _CH, _CH)], rows, vsem)

    def drain(iv, rows, vsem):
      pltpu.make_async_copy(vals_hbm.at[pl.ds(base, _CH)], rows, vsem).wait()
      pltpu.sync_copy(rows, acc.at[iv], add=True)

    issue(0, i0, r0, v0)

    def body(t, carry):
      j1 = 2 * t + 1
      j2 = j1 + 1
      issue(j1, i1, r1, v1)
      drain(i0, r0, v0)
      issue(j2, i0, r0, v0)
      drain(i1, r1, v1)
      return carry

    lax.fori_loop(0, _NPAIR, body, 0)
    drain(i0, r0, v0)
    plsc.subcore_barrier()

    @pl.when(c == 0)
    def _():
      pltpu.sync_copy(acc.at[pl.ds(s * _RPS, _RPS)],
                      out0_hbm.at[pl.ds(s * _RPS, _RPS)])

    @pl.when(c == 1)
    def _():
      pltpu.sync_copy(acc.at[pl.ds(s * _RPS, _RPS)],
                      out1_hbm.at[pl.ds(s * _RPS, _RPS)])

  return k(vals, dst, zeros_n)


def _sc_gather_sub(ntab, g, src, rev):
  @functools.partial(
      pl.kernel,
      out_type=jax.ShapeDtypeStruct((_E, _H), jnp.float32),
      mesh=_sc_mesh(),
      scratch_types=[
          pltpu.VMEM((_EPW,), jnp.int32),
          pltpu.VMEM((_EPW,), jnp.int32),
          pltpu.VMEM((_CH, _H), jnp.float32),
          pltpu.VMEM((_CH, _H), jnp.float32),
          pltpu.VMEM((_CH, _H), jnp.float32),
          pltpu.VMEM((_CH, _H), jnp.float32),
          pltpu.VMEM((_CH, _H), jnp.float32),
          pltpu.VMEM((_CH, _H), jnp.float32),
          pltpu.SemaphoreType.DMA,
          pltpu.SemaphoreType.DMA,
          pltpu.SemaphoreType.DMA,
          pltpu.SemaphoreType.DMA,
      ],
  )
  def k(ntab_hbm, g_hbm, src_hbm, rev_hbm, out_hbm,
        src_all, rev_all, ns0, gr0, ns1, gr1, d0, d1, g0, g1, s0, s1):
    base = _worker_id() * _EPW
    pltpu.sync_copy(src_hbm.at[pl.ds(base, _EPW)], src_all)
    pltpu.sync_copy(rev_hbm.at[pl.ds(base, _EPW)], rev_all)

    def issue(j, nsb, grb, gsem):
      pltpu.async_copy(ntab_hbm.at[src_all.at[pl.ds(j * _CH, _CH)]],
                       nsb, gsem)
      pltpu.async_copy(g_hbm.at[rev_all.at[pl.ds(j * _CH, _CH)]], grb, gsem)

    def wait_g(nsb, grb, gsem):
      pltpu.make_async_copy(ntab_hbm.at[src_all.at[pl.ds(0, _CH)]],
                            nsb, gsem).wait()
      pltpu.make_async_copy(g_hbm.at[rev_all.at[pl.ds(0, _CH)]],
                            grb, gsem).wait()

    def alu(nsb, grb, db):
      def row(r, carry):
        for cc in range(_NCOL):
          sl = pl.ds(cc * 16, 16)
          db[r, sl] = nsb[r, sl] - grb[r, sl]
        return carry
      lax.fori_loop(0, _CH, row, 0)

    def wait_s(db, ssem):
      pltpu.make_async_copy(db, out_hbm.at[pl.ds(base, _CH)], ssem).wait()

    issue(0, ns0, gr0, g0)

    def body(t, carry):
      j0 = 2 * t
      j1 = j0 + 1
      j2 = j0 + 2
      issue(j1, ns1, gr1, g1)
      wait_g(ns0, gr0, g0)

      @pl.when(t > 0)
      def _():
        wait_s(d0, s0)

      alu(ns0, gr0, d0)
      pltpu.async_copy(d0, out_hbm.at[pl.ds(base + j0 * _CH, _CH)], s0)
      issue(j2, ns0, gr0, g0)
      wait_g(ns1, gr1, g1)

      @pl.when(t > 0)
      def _():
        wait_s(d1, s1)

      alu(ns1, gr1, d1)
      pltpu.async_copy(d1, out_hbm.at[pl.ds(base + j1 * _CH, _CH)], s1)
      return carry

    lax.fori_loop(0, _NPAIR, body, 0)
    wait_g(ns0, gr0, g0)
    wait_s(d0, s0)
    alu(ns0, gr0, d0)
    pltpu.sync_copy(d0, out_hbm.at[pl.ds(base + (_NCH - 1) * _CH, _CH)])
    wait_s(d1, s1)

  return k(ntab, g, src, rev)


_BE = 2560
_GE = _E // _BE


def _tc_h0(xs, ea, w1t):
  def body(xs_ref, ea_ref, w1t_ref, h0_ref):
    xse = jnp.concatenate([xs_ref[...], ea_ref[...]], axis=1)
    h0_ref[...] = jnp.maximum(
        jnp.dot(xse, w1t_ref[...], preferred_element_type=jnp.float32), 0.0)

  return pl.pallas_call(
      body,
      grid=(_GE,),
      in_specs=[
          pl.BlockSpec((_BE, _NF), lambda i: (i, 0)),
          pl.BlockSpec((_BE, 1), lambda i: (i, 0)),
          pl.BlockSpec((_NF + 1, _H), lambda i: (0, 0)),
      ],
      out_specs=pl.BlockSpec((_BE, _H), lambda i: (i, 0)),
      out_shape=jax.ShapeDtypeStruct((_E, _H), jnp.float32),
  )(xs, ea, w1t)


def _tc_layer(h0, m, w2t):
  def body(h0_ref, m_ref, w2t_ref, h_ref):
    h_ref[...] = jnp.maximum(
        h0_ref[...]
        + jnp.dot(m_ref[...], w2t_ref[...], preferred_element_type=jnp.float32),
        0.0)

  return pl.pallas_call(
      body,
      grid=(_GE,),
      in_specs=[
          pl.BlockSpec((_BE, _H), lambda i: (i, 0)),
          pl.BlockSpec((_BE, _H), lambda i: (i, 0)),
          pl.BlockSpec((_H, _H), lambda i: (0, 0)),
      ],
      out_specs=pl.BlockSpec((_BE, _H), lambda i: (i, 0)),
      out_shape=jax.ShapeDtypeStruct((_E, _H), jnp.float32),
  )(h0, m, w2t)


_BN = 1280
_GN = _NP // _BN


def _tc_add(a, b):
  def body(a_ref, b_ref, o_ref):
    o_ref[...] = a_ref[...] + b_ref[...]

  return pl.pallas_call(
      body,
      grid=(_GN,),
      in_specs=[
          pl.BlockSpec((_BN, _H), lambda i: (i, 0)),
          pl.BlockSpec((_BN, _H), lambda i: (i, 0)),
      ],
      out_specs=pl.BlockSpec((_BN, _H), lambda i: (i, 0)),
      out_shape=jax.ShapeDtypeStruct((_NP, _H), jnp.float32),
  )(a, b)


_BF = 1000
_GF = _N // _BF


def _tc_final(x, vp0, vp1, batch2d, w3t, b3row, fcwt, fcbrow):
  def body(x_ref, vp0_ref, vp1_ref, b_ref, w3t_ref, b3_ref,
           fcwt_ref, fcb_ref, out_ref, acc_s, acc_c):
    i = pl.program_id(0)
    vmsg = vp0_ref[...] + vp1_ref[...]
    z = jnp.concatenate([x_ref[...], vmsg], axis=1)
    na = jnp.maximum(
        jnp.dot(z, w3t_ref[...], preferred_element_type=jnp.float32)
        + b3_ref[...], 0.0)
    cols = lax.broadcasted_iota(jnp.int32, (_BF, _B), 1)
    oh = (b_ref[...] == cols).astype(jnp.float32)
    ps = lax.dot_general(oh, na, (((0,), (0,)), ((), ())),
                         preferred_element_type=jnp.float32,
                         precision=lax.Precision.HIGHEST)
    pc = lax.dot_general(oh, jnp.ones((_BF, _H), jnp.float32),
                         (((0,), (0,)), ((), ())),
                         preferred_element_type=jnp.float32,
                         precision=lax.Precision.HIGHEST)

    @pl.when(i == 0)
    def _():
      acc_s[...] = jnp.zeros((_B, _H), jnp.float32)
      acc_c[...] = jnp.zeros((_B, _H), jnp.float32)

    acc_s[...] += ps
    acc_c[...] += pc

    @pl.when(i == _GF - 1)
    def _():
      pooled = acc_s[...] / jnp.maximum(acc_c[...], 1.0)
      out_ref[...] = jnp.tanh(
          jnp.dot(pooled, fcwt_ref[...], preferred_element_type=jnp.float32)
          + fcb_ref[...])

  return pl.pallas_call(
      body,
      grid=(_GF,),
      in_specs=[
          pl.BlockSpec((_BF, _NF), lambda i: (i, 0)),
          pl.BlockSpec((_BF, _H), lambda i: (i, 0)),
          pl.BlockSpec((_BF, _H), lambda i: (i, 0)),
          pl.BlockSpec((_BF, 1), lambda i: (i, 0)),
          pl.BlockSpec((_NF + _H, _H), lambda i: (0, 0)),
          pl.BlockSpec((1, _H), lambda i: (0, 0)),
          pl.BlockSpec((_H, _EMB), lambda i: (0, 0)),
          pl.BlockSpec((1, _EMB), lambda i: (0, 0)),
      ],
      out_specs=pl.BlockSpec((_B, _EMB), lambda i: (0, 0)),
      out_shape=jax.ShapeDtypeStruct((_B, _EMB), jnp.float32),
      scratch_shapes=[
          pltpu.VMEM((_B, _H), jnp.float32),
          pltpu.VMEM((_B, _H), jnp.float32),
      ],
  )(x, vp0, vp1, batch2d, w3t, b3row, fcwt, fcbrow)


def kernel(x, edge_index, revedge_index, edge_attr, batch, W1, W2, W3, b3,
           fcW, fcb):
  src = edge_index[0]
  dst = edge_index[1]
  ea = edge_attr[:, None]
  w1t = W1.T
  w2t = W2.T
  w3t = W3.T
  b3row = b3[None, :]
  fcwt = fcW.T
  fcbrow = fcb[None, :]
  zeros_n = jnp.zeros((_NP, _H), jnp.float32)
  batch2d = batch[:, None]

  xs = _sc_gather(x, src)
  h0 = _tc_h0(xs, ea, w1t)

  h = h0
  for _ in range(2):
    pa, pb = _sc_scatter_sum(h, dst, zeros_n)
    n = _tc_add(pa, pb)
    m = _sc_gather_sub(n, h, src, revedge_index)
    h = _tc_layer(h0, m, w2t)

  vpa, vpb = _sc_scatter_sum(h, dst, zeros_n)
  return _tc_final(x, vpa, vpb, batch2d, w3t, b3row, fcwt, fcbrow)


# speedup vs baseline: 3.4641x; 1.0505x over previous
import functools

import jax
import jax.numpy as jnp
from jax import lax
from jax.experimental import pallas as pl
from jax.experimental.pallas import tpu as pltpu
from jax.experimental.pallas import tpu_sc as plsc

_N = 10000
_E = 320000
_NF = 128
_H = 128
_EMB = 64
_B = 64

_NC = 2
_NS = 16
_NW = _NC * _NS
_EPW = _E // _NW
_CH = 80
_NCH = _EPW // _CH
_NPAIR = (_NCH - 1) // 2
_NP = 10240
_RPS = _NP // _NS
_NCOL = _H // 16


def _sc_mesh():
  return plsc.VectorSubcoreMesh(core_axis_name="c", subcore_axis_name="s")


def _worker_id():
  return lax.axis_index("s") * _NC + lax.axis_index("c")


def _sc_gather(table, idx):
  @functools.partial(
      pl.kernel,
      out_type=jax.ShapeDtypeStruct((_E, _H), jnp.float32),
      mesh=_sc_mesh(),
      scratch_types=[
          pltpu.VMEM((_EPW,), jnp.int32),
          pltpu.VMEM((_CH, _H), jnp.float32),
          pltpu.VMEM((_CH, _H), jnp.float32),
          pltpu.SemaphoreType.DMA,
          pltpu.SemaphoreType.DMA,
          pltpu.SemaphoreType.DMA,
          pltpu.SemaphoreType.DMA,
      ],
  )
  def k(table_hbm, idx_hbm, out_hbm, idx_all, r0, r1, g0, g1, s0, s1):
    base = _worker_id() * _EPW
    pltpu.sync_copy(idx_hbm.at[pl.ds(base, _EPW)], idx_all)

    def issue(j, rows, gsem):
      pltpu.async_copy(table_hbm.at[idx_all.at[pl.ds(j * _CH, _CH)]],
                       rows, gsem)

    def wait_g(rows, gsem):
      pltpu.make_async_copy(table_hbm.at[idx_all.at[pl.ds(0, _CH)]],
                            rows, gsem).wait()

    def store(j, rows, ssem):
      pltpu.async_copy(rows, out_hbm.at[pl.ds(base + j * _CH, _CH)], ssem)

    def wait_s(rows, ssem):
      pltpu.make_async_copy(rows, out_hbm.at[pl.ds(base, _CH)], ssem).wait()

    issue(0, r0, g0)

    def body(t, carry):
      j0 = 2 * t
      j1 = j0 + 1
      j2 = j0 + 2

      @pl.when(t > 0)
      def _():
        wait_s(r1, s1)

      issue(j1, r1, g1)
      wait_g(r0, g0)
      store(j0, r0, s0)
      wait_s(r0, s0)
      issue(j2, r0, g0)
      wait_g(r1, g1)
      store(j1, r1, s1)
      return carry

    lax.fori_loop(0, _NPAIR, body, 0)
    wait_g(r0, g0)
    pltpu.sync_copy(r0, out_hbm.at[pl.ds(base + (_NCH - 1) * _CH, _CH)])
    wait_s(r1, s1)

  return k(table, idx)


def _sc_scatter_sum(vals, dstp, zeros_n):
  @functools.partial(
      pl.kernel,
      out_type=(
          jax.ShapeDtypeStruct((_NP, _H), jnp.float32),
          jax.ShapeDtypeStruct((_NP, _H), jnp.float32),
      ),
      mesh=_sc_mesh(),
      scratch_types=[
          pltpu.VMEM((_NCH, _CH), jnp.int32),
          pltpu.VMEM((_CH, _H), jnp.float32),
          pltpu.VMEM((_CH, _H), jnp.float32),
          pltpu.VMEM_SHARED((_NP, _H), jnp.float32),
          pltpu.SemaphoreType.DMA,
          pltpu.SemaphoreType.DMA,
      ],
  )
  def k(vals_hbm, dstp_hbm, z_hbm, out0_hbm, out1_hbm,
        dstall, r0, r1, acc, v0, v1):
    c = lax.axis_index("c")
    s = lax.axis_index("s")
    w = s * _NC + c
    base = w * _EPW
    pltpu.sync_copy(z_hbm.at[pl.ds(s * _RPS, _RPS)],
                    acc.at[pl.ds(s * _RPS, _RPS)])
    pltpu.sync_copy(dstp_hbm.at[w], dstall)
    plsc.subcore_barrier()

    def issue(j, rows, vsem):
      pltpu.async_copy(vals_hbm.at[pl.ds(base + j * _CH, _CH)], rows, vsem)

    def drain(j, rows, vsem):
      pltpu.make_async_copy(vals_hbm.at[pl.ds(base, _CH)], rows, vsem).wait()
      pltpu.sync_copy(rows, acc.at[dstall.at[j]], add=True)

    issue(0, r0, v0)

    def body(t, carry):
      j0 = 2 * t
      j1 = j0 + 1
      j2 = j0 + 2
      issue(j1, r1, v1)
      drain(j0, r0, v0)
      issue(j2, r0, v0)
      drain(j1, r1, v1)
      return carry

    lax.fori_loop(0, _NPAIR, body, 0)
    drain(_NCH - 1, r0, v0)
    plsc.subcore_barrier()

    @pl.when(c == 0)
    def _():
      pltpu.sync_copy(acc.at[pl.ds(s * _RPS, _RPS)],
                      out0_hbm.at[pl.ds(s * _RPS, _RPS)])

    @pl.when(c == 1)
    def _():
      pltpu.sync_copy(acc.at[pl.ds(s * _RPS, _RPS)],
                      out1_hbm.at[pl.ds(s * _RPS, _RPS)])

  return k(vals, dstp, zeros_n)


def _sc_gather_sub(ntab, g, src, rev):
  @functools.partial(
      pl.kernel,
      out_type=jax.ShapeDtypeStruct((_E, _H), jnp.float32),
      mesh=_sc_mesh(),
      scratch_types=[
          pltpu.VMEM((_EPW,), jnp.int32),
          pltpu.VMEM((_EPW,), jnp.int32),
          pltpu.VMEM((_CH, _H), jnp.float32),
          pltpu.VMEM((_CH, _H), jnp.float32),
          pltpu.VMEM((_CH, _H), jnp.float32),
          pltpu.VMEM((_CH, _H), jnp.float32),
          pltpu.VMEM((_CH, _H), jnp.float32),
          pltpu.VMEM((_CH, _H), jnp.float32),
          pltpu.SemaphoreType.DMA,
          pltpu.SemaphoreType.DMA,
          pltpu.SemaphoreType.DMA,
          pltpu.SemaphoreType.DMA,
      ],
  )
  def k(ntab_hbm, g_hbm, src_hbm, rev_hbm, out_hbm,
        src_all, rev_all, ns0, gr0, ns1, gr1, d0, d1, g0, g1, s0, s1):
    base = _worker_id() * _EPW
    pltpu.sync_copy(src_hbm.at[pl.ds(base, _EPW)], src_all)
    pltpu.sync_copy(rev_hbm.at[pl.ds(base, _EPW)], rev_all)

    def issue(j, nsb, grb, gsem):
      pltpu.async_copy(ntab_hbm.at[src_all.at[pl.ds(j * _CH, _CH)]],
                       nsb, gsem)
      pltpu.async_copy(g_hbm.at[rev_all.at[pl.ds(j * _CH, _CH)]], grb, gsem)

    def wait_g(nsb, grb, gsem):
      pltpu.make_async_copy(ntab_hbm.at[src_all.at[pl.ds(0, _CH)]],
                            nsb, gsem).wait()
      pltpu.make_async_copy(g_hbm.at[rev_all.at[pl.ds(0, _CH)]],
                            grb, gsem).wait()

    def alu(nsb, grb, db):
      def row(r, carry):
        for cc in range(_NCOL):
          sl = pl.ds(cc * 16, 16)
          db[r, sl] = nsb[r, sl] - grb[r, sl]
        return carry
      lax.fori_loop(0, _CH, row, 0)

    def wait_s(db, ssem):
      pltpu.make_async_copy(db, out_hbm.at[pl.ds(base, _CH)], ssem).wait()

    issue(0, ns0, gr0, g0)

    def body(t, carry):
      j0 = 2 * t
      j1 = j0 + 1
      j2 = j0 + 2
      issue(j1, ns1, gr1, g1)
      wait_g(ns0, gr0, g0)

      @pl.when(t > 0)
      def _():
        wait_s(d0, s0)

      alu(ns0, gr0, d0)
      pltpu.async_copy(d0, out_hbm.at[pl.ds(base + j0 * _CH, _CH)], s0)
      issue(j2, ns0, gr0, g0)
      wait_g(ns1, gr1, g1)

      @pl.when(t > 0)
      def _():
        wait_s(d1, s1)

      alu(ns1, gr1, d1)
      pltpu.async_copy(d1, out_hbm.at[pl.ds(base + j1 * _CH, _CH)], s1)
      return carry

    lax.fori_loop(0, _NPAIR, body, 0)
    wait_g(ns0, gr0, g0)
    wait_s(d0, s0)
    alu(ns0, gr0, d0)
    pltpu.sync_copy(d0, out_hbm.at[pl.ds(base + (_NCH - 1) * _CH, _CH)])
    wait_s(d1, s1)

  return k(ntab, g, src, rev)


_BE = 2560
_GE = _E // _BE


def _tc_h0(xs, ea, w1t):
  def body(xs_ref, ea_ref, w1t_ref, h0_ref):
    xse = jnp.concatenate([xs_ref[...], ea_ref[...]], axis=1)
    h0_ref[...] = jnp.maximum(
        jnp.dot(xse, w1t_ref[...], preferred_element_type=jnp.float32), 0.0)

  return pl.pallas_call(
      body,
      grid=(_GE,),
      in_specs=[
          pl.BlockSpec((_BE, _NF), lambda i: (i, 0)),
          pl.BlockSpec((_BE, 1), lambda i: (i, 0)),
          pl.BlockSpec((_NF + 1, _H), lambda i: (0, 0)),
      ],
      out_specs=pl.BlockSpec((_BE, _H), lambda i: (i, 0)),
      out_shape=jax.ShapeDtypeStruct((_E, _H), jnp.float32),
  )(xs, ea, w1t)


def _tc_layer(h0, m, w2t):
  def body(h0_ref, m_ref, w2t_ref, h_ref):
    h_ref[...] = jnp.maximum(
        h0_ref[...]
        + jnp.dot(m_ref[...], w2t_ref[...], preferred_element_type=jnp.float32),
        0.0)

  return pl.pallas_call(
      body,
      grid=(_GE,),
      in_specs=[
          pl.BlockSpec((_BE, _H), lambda i: (i, 0)),
          pl.BlockSpec((_BE, _H), lambda i: (i, 0)),
          pl.BlockSpec((_H, _H), lambda i: (0, 0)),
      ],
      out_specs=pl.BlockSpec((_BE, _H), lambda i: (i, 0)),
      out_shape=jax.ShapeDtypeStruct((_E, _H), jnp.float32),
  )(h0, m, w2t)


_BN = 1280
_GN = _NP // _BN


def _tc_add(a, b):
  def body(a_ref, b_ref, o_ref):
    o_ref[...] = a_ref[...] + b_ref[...]

  return pl.pallas_call(
      body,
      grid=(_GN,),
      in_specs=[
          pl.BlockSpec((_BN, _H), lambda i: (i, 0)),
          pl.BlockSpec((_BN, _H), lambda i: (i, 0)),
      ],
      out_specs=pl.BlockSpec((_BN, _H), lambda i: (i, 0)),
      out_shape=jax.ShapeDtypeStruct((_NP, _H), jnp.float32),
  )(a, b)


_BF = 1000
_GF = _N // _BF


def _tc_final(x, vp0, vp1, batch2d, w3t, b3row, fcwt, fcbrow):
  def body(x_ref, vp0_ref, vp1_ref, b_ref, w3t_ref, b3_ref,
           fcwt_ref, fcb_ref, out_ref, acc_s, acc_c):
    i = pl.program_id(0)
    vmsg = vp0_ref[...] + vp1_ref[...]
    z = jnp.concatenate([x_ref[...], vmsg], axis=1)
    na = jnp.maximum(
        jnp.dot(z, w3t_ref[...], preferred_element_type=jnp.float32)
        + b3_ref[...], 0.0)
    cols = lax.broadcasted_iota(jnp.int32, (_BF, _B), 1)
    oh = (b_ref[...] == cols).astype(jnp.float32)
    ps = lax.dot_general(oh, na, (((0,), (0,)), ((), ())),
                         preferred_element_type=jnp.float32,
                         precision=lax.Precision.HIGHEST)
    pc = lax.dot_general(oh, jnp.ones((_BF, _H), jnp.float32),
                         (((0,), (0,)), ((), ())),
                         preferred_element_type=jnp.float32,
                         precision=lax.Precision.HIGHEST)

    @pl.when(i == 0)
    def _():
      acc_s[...] = jnp.zeros((_B, _H), jnp.float32)
      acc_c[...] = jnp.zeros((_B, _H), jnp.float32)

    acc_s[...] += ps
    acc_c[...] += pc

    @pl.when(i == _GF - 1)
    def _():
      pooled = acc_s[...] / jnp.maximum(acc_c[...], 1.0)
      out_ref[...] = jnp.tanh(
          jnp.dot(pooled, fcwt_ref[...], preferred_element_type=jnp.float32)
          + fcb_ref[...])

  return pl.pallas_call(
      body,
      grid=(_GF,),
      in_specs=[
          pl.BlockSpec((_BF, _NF), lambda i: (i, 0)),
          pl.BlockSpec((_BF, _H), lambda i: (i, 0)),
          pl.BlockSpec((_BF, _H), lambda i: (i, 0)),
          pl.BlockSpec((_BF, 1), lambda i: (i, 0)),
          pl.BlockSpec((_NF + _H, _H), lambda i: (0, 0)),
          pl.BlockSpec((1, _H), lambda i: (0, 0)),
          pl.BlockSpec((_H, _EMB), lambda i: (0, 0)),
          pl.BlockSpec((1, _EMB), lambda i: (0, 0)),
      ],
      out_specs=pl.BlockSpec((_B, _EMB), lambda i: (0, 0)),
      out_shape=jax.ShapeDtypeStruct((_B, _EMB), jnp.float32),
      scratch_shapes=[
          pltpu.VMEM((_B, _H), jnp.float32),
          pltpu.VMEM((_B, _H), jnp.float32),
      ],
  )(x, vp0, vp1, batch2d, w3t, b3row, fcwt, fcbrow)


def kernel(x, edge_index, revedge_index, edge_attr, batch, W1, W2, W3, b3,
           fcW, fcb):
  src = edge_index[0]
  dst = edge_index[1]
  ea = edge_attr[:, None]
  w1t = W1.T
  w2t = W2.T
  w3t = W3.T
  b3row = b3[None, :]
  fcwt = fcW.T
  fcbrow = fcb[None, :]
  zeros_n = jnp.zeros((_NP, _H), jnp.float32)
  batch2d = batch[:, None]
  dstp = dst.reshape(_NW, _NCH, _CH)

  xs = _sc_gather(x, src)
  h0 = _tc_h0(xs, ea, w1t)

  h = h0
  for _ in range(2):
    pa, pb = _sc_scatter_sum(h, dstp, zeros_n)
    n = _tc_add(pa, pb)
    m = _sc_gather_sub(n, h, src, revedge_index)
    h = _tc_layer(h0, m, w2t)

  vpa, vpb = _sc_scatter_sum(h, dstp, zeros_n)
  return _tc_final(x, vpa, vpb, batch2d, w3t, b3row, fcwt, fcbrow)
